# baseline (speedup 1.0000x reference)
import functools

import jax
import jax.numpy as jnp
from jax import lax
from jax.experimental import pallas as pl
from jax.experimental.pallas import tpu as pltpu
from jax.experimental.pallas import tpu_sc as plsc

VOCAB = 1000000
EMBED = 64
BATCH = 16384
CTX = 5

NC, NS, L = 2, 16, 16
NW = NC * NS
PER_W = BATCH // NW
CB = 128
CHUNKS = PER_W // CB
ROWS_C = CB * CTX
UNROLL = 4


def _body(ttab, ctab, tidx, cidx, out, t_idx_v, c_idx_v, t_rows, c_rows, out_v, sem):
    wid = lax.axis_index("s") * NC + lax.axis_index("c")
    iota = lax.iota(jnp.int32, L)
    iota5 = iota * CTX

    for g in range(CHUNKS):
        pltpu.sync_copy(tidx.at[pl.ds((wid * CHUNKS + g) * CB, CB)], t_idx_v)
        pltpu.sync_copy(cidx.at[pl.ds((wid * CHUNKS + g) * ROWS_C, ROWS_C)], c_idx_v)
        cps = [pltpu.async_copy(ttab.at[t_idx_v], t_rows, sem)]
        for j in range(CTX):
            cps.append(pltpu.async_copy(
                ctab.at[c_idx_v.at[pl.ds(j * CB, CB)]], c_rows.at[pl.ds(j * CB, CB)], sem))
        for cp in cps:
            cp.wait()

        zero = jnp.zeros((L,), jnp.float32)

        def blk_body(blk, _):
            boff = blk * L

            def b_body(i, res):
                b = boff + i
                tch = [t_rows[b, pl.ds(k * L, L)] for k in range(EMBED // L)]
                eq = iota == i
                new = []
                for c in range(CTX):
                    r = b * CTX + c
                    acc = tch[0] * c_rows[r, pl.ds(0, L)]
                    for k in range(1, EMBED // L):
                        acc = acc + tch[k] * c_rows[r, pl.ds(k * L, L)]
                    s = jnp.sum(acc)
                    new.append(jnp.where(eq, s, res[c]))
                return tuple(new)

            res = lax.fori_loop(0, L, b_body, (zero,) * CTX)
            for c in range(CTX):
                plsc.store_scatter(out_v, [iota5 + (boff * CTX + c)], res[c])
            return 0

        lax.fori_loop(0, CB // L, blk_body, 0)

        pltpu.sync_copy(out_v, out.at[pl.ds((wid * CHUNKS + g) * ROWS_C, ROWS_C)])


@jax.jit
def kernel(target, context, target_table, context_table):
    tidx = target.astype(jnp.int32).reshape(BATCH)
    cidx = context.astype(jnp.int32).reshape(BATCH * CTX)
    fn = pl.kernel(
        _body,
        out_type=jax.ShapeDtypeStruct((BATCH * CTX,), jnp.float32),
        mesh=plsc.VectorSubcoreMesh(core_axis_name="c", subcore_axis_name="s"),
        compiler_params=pltpu.CompilerParams(
            needs_layout_passes=False, use_tc_tiling_on_sc=False),
        scratch_types=[
            pltpu.VMEM((CB,), jnp.int32),
            pltpu.VMEM((ROWS_C,), jnp.int32),
            pltpu.VMEM((CB, EMBED), jnp.float32),
            pltpu.VMEM((ROWS_C, EMBED), jnp.float32),
            pltpu.VMEM((ROWS_C,), jnp.float32),
            pltpu.SemaphoreType.DMA,
        ],
    )
    dots = fn(target_table, context_table, tidx, cidx)
    return dots.reshape(BATCH, CTX)
